# baseline (speedup 1.0000x reference)
import dataclasses
import functools

import jax
import jax.numpy as jnp
from jax import lax
from jax.experimental import pallas as pl
from jax.experimental.pallas import tpu as pltpu
from jax.experimental.pallas import tpu_sc as plsc

N = 10000
E = 160000
DIN = 256
DH = 256
DOUT = 128
L = 3
G = 64
HALF = 128

NSUB = 16
EPT = E // NSUB
SUP = 2000
NSUP = EPT // SUP
CB = 80
NCHUNK = SUP // CB
NBUF = 3
NPAD = 10112
ROWS_PER_TILE = NPAD // NSUB


NFB = 1


def _edge_body(hbf_hbm, srcs_hbm, dst_hbm, a0_hbm, a1_hbm, w_hbm, zin_hbm,
               out_hbm, wvbf, gbuf, fbuf, isup, dsup, a0sup, a1sup, istage,
               dstage, aggsh, gat_sems, scat_sems):
    c = lax.axis_index("c")
    s = lax.axis_index("s")

    pltpu.sync_copy(w_hbm.at[c], wvbf)
    pltpu.sync_copy(zin_hbm, aggsh.at[pl.ds(s * ROWS_PER_TILE, ROWS_PER_TILE)])
    plsc.subcore_barrier()

    w0 = [plsc.bitcast(wvbf[0, pl.ds(16 * g, 16)], jnp.bfloat16)
          for g in range(4)]
    w1 = [plsc.bitcast(wvbf[1, pl.ds(16 * g, 16)], jnp.bfloat16)
          for g in range(4)]
    mask_hi = jnp.int32(-65536)
    zi16 = jnp.zeros((16,), jnp.int32)

    def stage(kk, b):
        for i in range(CB // 16):
            sl = pl.ds(kk * CB + 16 * i, 16)
            istage[b, pl.ds(16 * i, 16)] = isup[0, sl]

    def stage_d(kk, b):
        for i in range(CB // 16):
            sl = pl.ds(kk * CB + 16 * i, 16)
            dstage[b, pl.ds(16 * i, 16)] = dsup[0, sl]

    def gat_start(b):
        pltpu.async_copy(hbf_hbm.at[istage.at[b]], gbuf.at[b], gat_sems[b])

    def gat_wait(b):
        pltpu.make_async_copy(hbf_hbm.at[istage.at[b]], gbuf.at[b],
                              gat_sems[b]).wait()

    def scat_start(b):
        pltpu.async_copy(fbuf.at[b], aggsh.at[dstage.at[b]], scat_sems[b],
                         add=True)

    def scat_wait(b):
        pltpu.make_async_copy(fbuf.at[b], aggsh.at[dstage.at[b]],
                              scat_sems[b]).wait()

    def compute(kk, gb):
        cw = c * (HALF // 2)

        @pl.loop(0, CB // 16)
        def _(e16):
            eb = e16 * 16
            a0v = a0sup[0, pl.ds(kk * CB + eb, 16)]
            a1v = a1sup[0, pl.ds(kk * CB + eb, 16)]
            for t in range(16):
                a0s = plsc.bitcast(a0v[t] + zi16, jnp.bfloat16)
                a1s = plsc.bitcast(a1v[t] + zi16, jnp.bfloat16)
                for g in range(4):
                    vb = plsc.bitcast(gbuf[gb, eb + t,
                                           pl.ds(cw + 16 * g, 16)],
                                      jnp.bfloat16)
                    m = jnp.maximum(vb + a0s * w0[g] + a1s * w1[g],
                                    jnp.bfloat16(0))
                    vi = plsc.bitcast(m, jnp.int32)
                    lo = plsc.bitcast(lax.shift_left(vi, 16), jnp.float32)
                    hi = plsc.bitcast(jnp.bitwise_and(vi, mask_hi),
                                      jnp.float32)
                    fbuf[0, eb + t, pl.ds(32 * g, 16)] = lo
                    fbuf[0, eb + t, pl.ds(32 * g + 16, 16)] = hi

    @pl.loop(0, NSUP)
    def _(sup):
        soff = s * EPT + sup * SUP
        pltpu.sync_copy(srcs_hbm.at[pl.ds(soff, SUP)], isup.at[0])
        pltpu.sync_copy(dst_hbm.at[pl.ds(soff, SUP)], dsup.at[0])
        pltpu.sync_copy(a0_hbm.at[pl.ds(soff, SUP)], a0sup.at[0])
        pltpu.sync_copy(a1_hbm.at[pl.ds(soff, SUP)], a1sup.at[0])

        for q in range(NBUF):
            stage(q, q)
            gat_start(q)

        @pl.loop(0, NCHUNK - 1, step=NBUF)
        def _(k):
            for q in range(NBUF):
                kk = k + q
                gb = q
                nb = (q + 2) % NBUF
                gat_wait(gb)

                @pl.when(jnp.logical_and(kk + 2 >= NBUF, kk + 2 < NCHUNK))
                def _():
                    stage(kk + 2, nb)
                    gat_start(nb)

                @pl.when(kk >= 1)
                def _():
                    scat_wait(0)
                stage_d(kk, 0)
                compute(kk, gb)
                scat_start(0)

        kt = NCHUNK - 1
        gat_wait(kt % NBUF)
        scat_wait(0)
        stage_d(kt, 0)
        compute(kt, kt % NBUF)
        scat_start(0)
        scat_wait(0)

    plsc.subcore_barrier()
    pltpu.sync_copy(aggsh.at[pl.ds(s * ROWS_PER_TILE, ROWS_PER_TILE)],
                    out_hbm.at[c, pl.ds(s * ROWS_PER_TILE, ROWS_PER_TILE)])


@jax.jit
def _edge_agg(hpk, srcs, dst, a0, a1, wconst_bf, zin):
    mesh = plsc.VectorSubcoreMesh(core_axis_name="c", subcore_axis_name="s")
    cp = pltpu.CompilerParams()
    if "needs_layout_passes" in pltpu.CompilerParams.__dataclass_fields__:
        cp = dataclasses.replace(cp, needs_layout_passes=False)
    return pl.kernel(
        _edge_body,
        out_type=jax.ShapeDtypeStruct((2, NPAD, HALF), jnp.float32),
        compiler_params=cp,
        mesh=mesh,
        scratch_types=[
            pltpu.VMEM((2, HALF // 2), jnp.int32),
            pltpu.VMEM((NBUF, CB, HALF), jnp.int32),
            pltpu.VMEM((NFB, CB, HALF), jnp.float32),
            pltpu.VMEM((1, SUP), jnp.int32),
            pltpu.VMEM((1, SUP), jnp.int32),
            pltpu.VMEM((1, SUP), jnp.int32),
            pltpu.VMEM((1, SUP), jnp.int32),
            pltpu.VMEM((NBUF, CB), jnp.int32),
            pltpu.VMEM((NFB, CB), jnp.int32),
            pltpu.VMEM_SHARED((NPAD, HALF), jnp.float32),
            [pltpu.SemaphoreType.DMA] * NBUF,
            [pltpu.SemaphoreType.DMA] * NFB,
        ],
    )(hpk, srcs, dst, a0, a1, wconst_bf, zin)


BR = 2000


def _pack_rows(hb):
    outs = []
    for ch in range(2):
        base = ch * HALF
        words = []
        for g in range(4):
            lo = hb[:, base + 32 * g: base + 32 * g + 16]
            hi = hb[:, base + 32 * g + 16: base + 32 * g + 32]
            lou = (lax.bitcast_convert_type(lo, jnp.uint32)
                   + jnp.uint32(0x8000)) >> jnp.uint32(16)
            hiu = (lax.bitcast_convert_type(hi, jnp.uint32)
                   + jnp.uint32(0x8000)) >> jnp.uint32(16)
            words.append((hiu << jnp.uint32(16)) | lou)
        outs.append(lax.bitcast_convert_type(
            jnp.concatenate(words, axis=1), jnp.int32))
    return jnp.concatenate(outs, axis=1)


def _layer_kernel(first, has_next, h_ref, a_ref, eps_ref, w1_ref, b1_ref,
                  w2_ref, b2_ref, w3_ref, b3_ref, wbn_ref, out_ref, opk_ref):
    h = jnp.concatenate([h_ref[0], h_ref[1]], axis=1)
    agg = jnp.concatenate([a_ref[0], a_ref[1]], axis=1)
    u = (1.0 + eps_ref[0, 0]) * h + agg
    t = jnp.maximum(jnp.dot(u, w1_ref[...],
                            preferred_element_type=jnp.float32) + b1_ref[...], 0.0)
    t = jnp.maximum(jnp.dot(t, w2_ref[...],
                            preferred_element_type=jnp.float32) + b2_ref[...], 0.0)
    t = jnp.dot(t, w3_ref[...], preferred_element_type=jnp.float32) + b3_ref[...]
    t = jnp.maximum(t, 0.0)
    if not first:
        t = t + h
    out_ref[0] = t[:, :HALF]
    out_ref[1] = t[:, HALF:]
    if has_next:
        opk_ref[...] = _pack_rows(t + wbn_ref[...])


@functools.partial(jax.jit, static_argnums=(2, 3))
def _layer_tc(h2, agg2, first, has_next, eps, w1, b1, w2, b2, w3, b3, wbn):
    grid = (N // BR,)
    bs_w = pl.BlockSpec((DH, DH), lambda i: (0, 0))
    bs_b = pl.BlockSpec((1, DH), lambda i: (0, 0))
    return pl.pallas_call(
        functools.partial(_layer_kernel, first, has_next),
        grid=grid,
        in_specs=[
            pl.BlockSpec((2, BR, HALF), lambda i: (0, i, 0)),
            pl.BlockSpec((2, BR, HALF), lambda i: (0, i, 0)),
            pl.BlockSpec((1, 1), lambda i: (0, 0)),
            bs_w, bs_b, bs_w, bs_b, bs_w, bs_b, bs_b,
        ],
        out_specs=[
            pl.BlockSpec((2, BR, HALF), lambda i: (0, i, 0)),
            pl.BlockSpec((BR, HALF), lambda i: (i, 0)),
        ],
        out_shape=[
            jax.ShapeDtypeStruct((2, N, HALF), jnp.float32),
            jax.ShapeDtypeStruct((N, HALF), jnp.int32),
        ],
    )(h2, agg2, eps, w1, b1, w2, b2, w3, b3, wbn)


def _prep_kernel(x_ref, wb_ref, out_ref, opk_ref):
    xb = x_ref[...]
    out_ref[0] = xb[:, :HALF]
    out_ref[1] = xb[:, HALF:]
    opk_ref[...] = _pack_rows(xb + wb_ref[...])


@jax.jit
def _prep_tc(x, wb0):
    grid = (N // BR,)
    return pl.pallas_call(
        _prep_kernel,
        grid=grid,
        in_specs=[
            pl.BlockSpec((BR, DIN), lambda i: (i, 0)),
            pl.BlockSpec((1, DIN), lambda i: (0, 0)),
        ],
        out_specs=[
            pl.BlockSpec((2, BR, HALF), lambda i: (0, i, 0)),
            pl.BlockSpec((BR, HALF), lambda i: (i, 0)),
        ],
        out_shape=[
            jax.ShapeDtypeStruct((2, N, HALF), jnp.float32),
            jax.ShapeDtypeStruct((N, HALF), jnp.int32),
        ],
    )(x, wb0)


def _finale_kernel(r0_ref, r1_ref, r2_ref, r3_ref, batch_ref, ra_ref, rc_ref,
                   rb_ref, rd_ref, out_ref, pool_acc, cnt_acc):
    i = pl.program_id(0)

    @pl.when(i == 0)
    def _():
        pool_acc[...] = jnp.zeros_like(pool_acc)
        cnt_acc[...] = jnp.zeros_like(cnt_acc)

    bvec = batch_ref[0]
    gids = lax.broadcasted_iota(jnp.int32, (G, BR), 0)
    oht = (gids == jnp.broadcast_to(bvec, (G, BR))).astype(jnp.float32)
    cnt_acc[...] += jnp.dot(oht, jnp.ones((BR, HALF), jnp.float32),
                            preferred_element_type=jnp.float32)
    for r, ref in enumerate((r0_ref, r1_ref, r2_ref, r3_ref)):
        rep = jnp.concatenate([ref[0], ref[1]], axis=1)
        pool_acc[r] += jnp.dot(oht, rep, preferred_element_type=jnp.float32)

    @pl.when(i == pl.num_programs(0) - 1)
    def _():
        scale_h = lax.rsqrt(jnp.maximum(cnt_acc[...], 1.0))
        scale = jnp.concatenate([scale_h, scale_h], axis=1)
        z = jnp.zeros((G, DOUT), jnp.float32)
        for r in range(4):
            p = pool_acc[r] * scale
            t = jnp.maximum(jnp.dot(p, ra_ref[r],
                                    preferred_element_type=jnp.float32)
                            + rc_ref[r], 0.0)
            z = z + jnp.dot(t, rb_ref[r],
                            preferred_element_type=jnp.float32) + rd_ref[r]
        out_ref[...] = z


@jax.jit
def _finale_tc(r0, r1, r2, r3, batch3, ra, rc, rb, rd):
    grid = (N // BR,)
    bs_rep = pl.BlockSpec((2, BR, HALF), lambda i: (0, i, 0))
    return pl.pallas_call(
        _finale_kernel,
        grid=grid,
        in_specs=[
            bs_rep, bs_rep, bs_rep, bs_rep,
            pl.BlockSpec((1, 1, BR), lambda i: (i, 0, 0)),
            pl.BlockSpec((4, DH, DH), lambda i: (0, 0, 0)),
            pl.BlockSpec((4, 1, DH), lambda i: (0, 0, 0)),
            pl.BlockSpec((4, DH, DOUT), lambda i: (0, 0, 0)),
            pl.BlockSpec((4, 1, DOUT), lambda i: (0, 0, 0)),
        ],
        out_specs=pl.BlockSpec((G, DOUT), lambda i: (0, 0)),
        out_shape=jax.ShapeDtypeStruct((G, DOUT), jnp.float32),
        scratch_shapes=[
            pltpu.VMEM((4, G, DH), jnp.float32),
            pltpu.VMEM((G, HALF), jnp.float32),
        ],
    )(r0, r1, r2, r3, batch3, ra, rc, rb, rd)


import numpy as _np

def _wpack(wrow):
    r = (lax.bitcast_convert_type(wrow, jnp.uint32)
         + jnp.uint32(0x8000)) >> jnp.uint32(16)
    rr = r.reshape(4, 2, 16)
    return lax.bitcast_convert_type(
        (rr[:, 1, :] << jnp.uint32(16)) | rr[:, 0, :], jnp.int32).reshape(64)


def kernel(x, edge_index, edge_attr, batch, params):
    src = edge_index[0].astype(jnp.int32)
    dst = edge_index[1].astype(jnp.int32)
    def _attr_pack(a):
        r = (lax.bitcast_convert_type(a, jnp.uint32)
             + jnp.uint32(0x8000)) >> jnp.uint32(16)
        return lax.bitcast_convert_type((r << jnp.uint32(16)) | r, jnp.int32)

    a0 = _attr_pack(edge_attr[:, 0])
    a1 = _attr_pack(edge_attr[:, 1])
    zin = jnp.zeros((ROWS_PER_TILE, HALF), jnp.float32)
    batch3 = batch.astype(jnp.int32).reshape(N // BR, 1, BR)

    wb0 = params['convs'][0]['lin_edge'][1].reshape(1, DIN)
    h2, pk = _prep_tc(x, wb0)
    reps = [h2]
    for i in range(L):
        cp = params['convs'][i]
        Wl, _ = cp['lin_edge']
        wbf = jnp.stack([
            jnp.stack([_wpack(Wl[0, :HALF]), _wpack(Wl[1, :HALF])]),
            jnp.stack([_wpack(Wl[0, HALF:]), _wpack(Wl[1, HALF:])]),
        ])
        agg2 = _edge_agg(pk, src, dst, a0, a1, wbf, zin)
        (W1, b1), (W2, b2), (W3, b3) = cp['mlp']
        has_next = i < L - 1
        wbn = (params['convs'][i + 1]['lin_edge'][1] if has_next
               else jnp.zeros((DH,), jnp.float32)).reshape(1, DH)
        h2, pk = _layer_tc(h2, agg2, i == 0, has_next, cp['eps'].reshape(1, 1),
                           W1, b1.reshape(1, DH), W2, b2.reshape(1, DH),
                           W3, b3.reshape(1, DH), wbn)
        reps.append(h2)

    ra = jnp.stack([params['readouts'][i][0][0] for i in range(4)])
    rc = jnp.stack([params['readouts'][i][0][1].reshape(1, DH) for i in range(4)])
    rb = jnp.stack([params['readouts'][i][1][0] for i in range(4)])
    rd = jnp.stack([params['readouts'][i][1][1].reshape(1, DOUT) for i in range(4)])
    return _finale_tc(reps[0], reps[1], reps[2], reps[3], batch3, ra, rc, rb, rd)


# speedup vs baseline: 2.1512x; 2.1512x over previous
import dataclasses
import functools

import jax
import jax.numpy as jnp
from jax import lax
from jax.experimental import pallas as pl
from jax.experimental.pallas import tpu as pltpu
from jax.experimental.pallas import tpu_sc as plsc

N = 10000
E = 160000
DIN = 256
DH = 256
DOUT = 128
L = 3
G = 64
HALF = 128

NSUB = 16
EPT = E // NSUB
SUP = 2000
NSUP = EPT // SUP
CB = 80
NCHUNK = SUP // CB
NBUF = 3
NPAD = 10112
ROWS_PER_TILE = NPAD // NSUB


def _edge_body(hbf_hbm, srcs_hbm, dst_hbm, a0_hbm, a1_hbm, w_hbm, zin_hbm,
               out_hbm, wvbf, sbuf, isup, dsup, a0sup, a1sup, istage,
               dstage, aggsh, gat_sems, scat_sems):
    c = lax.axis_index("c")
    s = lax.axis_index("s")

    pltpu.sync_copy(w_hbm.at[c], wvbf)
    pltpu.sync_copy(zin_hbm, aggsh.at[pl.ds(s * ROWS_PER_TILE, ROWS_PER_TILE)])
    plsc.subcore_barrier()

    w0 = [plsc.bitcast(wvbf[0, pl.ds(16 * g, 16)], jnp.bfloat16)
          for g in range(4)]
    w1 = [plsc.bitcast(wvbf[1, pl.ds(16 * g, 16)], jnp.bfloat16)
          for g in range(4)]
    mask_hi = jnp.int32(-65536)
    zi16 = jnp.zeros((16,), jnp.int32)

    def stage(kk, b):
        for i in range(CB // 16):
            sl = pl.ds(kk * CB + 16 * i, 16)
            istage[b, pl.ds(16 * i, 16)] = isup[0, sl]

    def stage_d(kk, b):
        for i in range(CB // 16):
            sl = pl.ds(kk * CB + 16 * i, 16)
            dstage[b, pl.ds(16 * i, 16)] = dsup[0, sl]

    def gat_start(b):
        pltpu.async_copy(hbf_hbm.at[istage.at[b]], sbuf.at[b], gat_sems[b])

    def gat_wait(b):
        pltpu.make_async_copy(hbf_hbm.at[istage.at[b]], sbuf.at[b],
                              gat_sems[b]).wait()

    def scat_start(b):
        pltpu.async_copy(sbuf.at[b], aggsh.at[dstage.at[b]], scat_sems[b],
                         add=True)

    def scat_wait(b):
        pltpu.make_async_copy(sbuf.at[b], aggsh.at[dstage.at[b]],
                              scat_sems[b]).wait()

    def compute(kk, b):
        cw = c * (HALF // 2)

        @pl.loop(0, CB // 16)
        def _(e16):
            eb = e16 * 16
            a0v = a0sup[0, pl.ds(kk * CB + eb, 16)]
            a1v = a1sup[0, pl.ds(kk * CB + eb, 16)]
            for t in range(16):
                a0s = plsc.bitcast(a0v[t] + zi16, jnp.bfloat16)
                a1s = plsc.bitcast(a1v[t] + zi16, jnp.bfloat16)
                vbs = [plsc.bitcast(sbuf[b, eb + t, pl.ds(cw + 16 * g, 16)],
                                    jnp.bfloat16) for g in range(4)]
                for g in range(4):
                    m = jnp.maximum(vbs[g] + a0s * w0[g] + a1s * w1[g],
                                    jnp.bfloat16(0))
                    vi = plsc.bitcast(m, jnp.int32)
                    lo = lax.shift_left(vi, 16)
                    hi = jnp.bitwise_and(vi, mask_hi)
                    sbuf[b, eb + t, pl.ds(32 * g, 16)] = plsc.bitcast(
                        lo, jnp.float32)
                    sbuf[b, eb + t, pl.ds(32 * g + 16, 16)] = plsc.bitcast(
                        hi, jnp.float32)

    @pl.loop(0, NSUP)
    def _(sup):
        soff = s * EPT + sup * SUP
        pltpu.sync_copy(srcs_hbm.at[pl.ds(soff, SUP)], isup.at[0])
        pltpu.sync_copy(dst_hbm.at[pl.ds(soff, SUP)], dsup.at[0])
        pltpu.sync_copy(a0_hbm.at[pl.ds(soff, SUP)], a0sup.at[0])
        pltpu.sync_copy(a1_hbm.at[pl.ds(soff, SUP)], a1sup.at[0])

        for q in range(NBUF):
            stage(q, q)
            stage_d(q, q)
            gat_start(q)

        @pl.loop(0, NCHUNK - 1, step=NBUF)
        def _(k):
            for q in range(NBUF):
                kk = k + q
                b = q
                gat_wait(b)
                compute(kk, b)
                scat_start(b)
                nb = (q + 2) % NBUF

                @pl.when(jnp.logical_and(kk + 2 >= NBUF, kk + 2 < NCHUNK))
                def _():
                    scat_wait(nb)
                    stage(kk + 2, nb)
                    stage_d(kk + 2, nb)
                    gat_start(nb)

        bt = (NCHUNK - 1) % NBUF
        gat_wait(bt)
        compute(NCHUNK - 1, bt)
        scat_start(bt)
        for q in range(NBUF):
            scat_wait(q)

    plsc.subcore_barrier()
    pltpu.sync_copy(aggsh.at[pl.ds(s * ROWS_PER_TILE, ROWS_PER_TILE)],
                    out_hbm.at[c, pl.ds(s * ROWS_PER_TILE, ROWS_PER_TILE)])


@jax.jit
def _edge_agg(hpk, srcs, dst, a0, a1, wconst_bf, zin):
    mesh = plsc.VectorSubcoreMesh(core_axis_name="c", subcore_axis_name="s")
    cp = pltpu.CompilerParams()
    if "needs_layout_passes" in pltpu.CompilerParams.__dataclass_fields__:
        cp = dataclasses.replace(cp, needs_layout_passes=False)
    return pl.kernel(
        _edge_body,
        out_type=jax.ShapeDtypeStruct((2, NPAD, HALF), jnp.float32),
        compiler_params=cp,
        mesh=mesh,
        scratch_types=[
            pltpu.VMEM((2, HALF // 2), jnp.int32),
            pltpu.VMEM((NBUF, CB, HALF), jnp.float32),
            pltpu.VMEM((1, SUP), jnp.int32),
            pltpu.VMEM((1, SUP), jnp.int32),
            pltpu.VMEM((1, SUP), jnp.int32),
            pltpu.VMEM((1, SUP), jnp.int32),
            pltpu.VMEM((NBUF, CB), jnp.int32),
            pltpu.VMEM((NBUF, CB), jnp.int32),
            pltpu.VMEM_SHARED((NPAD, HALF), jnp.float32),
            [pltpu.SemaphoreType.DMA] * NBUF,
            [pltpu.SemaphoreType.DMA] * NBUF,
        ],
    )(hpk, srcs, dst, a0, a1, wconst_bf, zin)


BR = 2000


def _pack_rows(hb):
    outs = []
    for ch in range(2):
        base = ch * HALF
        words = []
        for g in range(4):
            lo = hb[:, base + 32 * g: base + 32 * g + 16]
            hi = hb[:, base + 32 * g + 16: base + 32 * g + 32]
            lou = (lax.bitcast_convert_type(lo, jnp.uint32)
                   + jnp.uint32(0x8000)) >> jnp.uint32(16)
            hiu = (lax.bitcast_convert_type(hi, jnp.uint32)
                   + jnp.uint32(0x8000)) >> jnp.uint32(16)
            words.append((hiu << jnp.uint32(16)) | lou)
        outs.append(lax.bitcast_convert_type(
            jnp.concatenate(words, axis=1), jnp.int32))
    return jnp.concatenate(outs, axis=1)


def _layer_kernel(first, has_next, h_ref, a_ref, eps_ref, w1_ref, b1_ref,
                  w2_ref, b2_ref, w3_ref, b3_ref, wbn_ref, out_ref, opk_ref):
    h = jnp.concatenate([h_ref[0], h_ref[1]], axis=1)
    agg = jnp.concatenate([a_ref[0], a_ref[1]], axis=1)
    u = (1.0 + eps_ref[0, 0]) * h + agg
    t = jnp.maximum(jnp.dot(u, w1_ref[...],
                            preferred_element_type=jnp.float32) + b1_ref[...], 0.0)
    t = jnp.maximum(jnp.dot(t, w2_ref[...],
                            preferred_element_type=jnp.float32) + b2_ref[...], 0.0)
    t = jnp.dot(t, w3_ref[...], preferred_element_type=jnp.float32) + b3_ref[...]
    t = jnp.maximum(t, 0.0)
    if not first:
        t = t + h
    out_ref[0] = t[:, :HALF]
    out_ref[1] = t[:, HALF:]
    if has_next:
        opk_ref[...] = _pack_rows(t + wbn_ref[...])


@functools.partial(jax.jit, static_argnums=(2, 3))
def _layer_tc(h2, agg2, first, has_next, eps, w1, b1, w2, b2, w3, b3, wbn):
    grid = (N // BR,)
    bs_w = pl.BlockSpec((DH, DH), lambda i: (0, 0))
    bs_b = pl.BlockSpec((1, DH), lambda i: (0, 0))
    return pl.pallas_call(
        functools.partial(_layer_kernel, first, has_next),
        grid=grid,
        in_specs=[
            pl.BlockSpec((2, BR, HALF), lambda i: (0, i, 0)),
            pl.BlockSpec((2, BR, HALF), lambda i: (0, i, 0)),
            pl.BlockSpec((1, 1), lambda i: (0, 0)),
            bs_w, bs_b, bs_w, bs_b, bs_w, bs_b, bs_b,
        ],
        out_specs=[
            pl.BlockSpec((2, BR, HALF), lambda i: (0, i, 0)),
            pl.BlockSpec((BR, HALF), lambda i: (i, 0)),
        ],
        out_shape=[
            jax.ShapeDtypeStruct((2, N, HALF), jnp.float32),
            jax.ShapeDtypeStruct((N, HALF), jnp.int32),
        ],
    )(h2, agg2, eps, w1, b1, w2, b2, w3, b3, wbn)


def _prep_kernel(x_ref, wb_ref, out_ref, opk_ref):
    xb = x_ref[...]
    out_ref[0] = xb[:, :HALF]
    out_ref[1] = xb[:, HALF:]
    opk_ref[...] = _pack_rows(xb + wb_ref[...])


@jax.jit
def _prep_tc(x, wb0):
    grid = (N // BR,)
    return pl.pallas_call(
        _prep_kernel,
        grid=grid,
        in_specs=[
            pl.BlockSpec((BR, DIN), lambda i: (i, 0)),
            pl.BlockSpec((1, DIN), lambda i: (0, 0)),
        ],
        out_specs=[
            pl.BlockSpec((2, BR, HALF), lambda i: (0, i, 0)),
            pl.BlockSpec((BR, HALF), lambda i: (i, 0)),
        ],
        out_shape=[
            jax.ShapeDtypeStruct((2, N, HALF), jnp.float32),
            jax.ShapeDtypeStruct((N, HALF), jnp.int32),
        ],
    )(x, wb0)


def _finale_kernel(r0_ref, r1_ref, r2_ref, r3_ref, batch_ref, ra_ref, rc_ref,
                   rb_ref, rd_ref, out_ref, pool_acc, cnt_acc):
    i = pl.program_id(0)

    @pl.when(i == 0)
    def _():
        pool_acc[...] = jnp.zeros_like(pool_acc)
        cnt_acc[...] = jnp.zeros_like(cnt_acc)

    bvec = batch_ref[0]
    gids = lax.broadcasted_iota(jnp.int32, (G, BR), 0)
    oht = (gids == jnp.broadcast_to(bvec, (G, BR))).astype(jnp.float32)
    cnt_acc[...] += jnp.dot(oht, jnp.ones((BR, HALF), jnp.float32),
                            preferred_element_type=jnp.float32)
    for r, ref in enumerate((r0_ref, r1_ref, r2_ref, r3_ref)):
        rep = jnp.concatenate([ref[0], ref[1]], axis=1)
        pool_acc[r] += jnp.dot(oht, rep, preferred_element_type=jnp.float32)

    @pl.when(i == pl.num_programs(0) - 1)
    def _():
        scale_h = lax.rsqrt(jnp.maximum(cnt_acc[...], 1.0))
        scale = jnp.concatenate([scale_h, scale_h], axis=1)
        z = jnp.zeros((G, DOUT), jnp.float32)
        for r in range(4):
            p = pool_acc[r] * scale
            t = jnp.maximum(jnp.dot(p, ra_ref[r],
                                    preferred_element_type=jnp.float32)
                            + rc_ref[r], 0.0)
            z = z + jnp.dot(t, rb_ref[r],
                            preferred_element_type=jnp.float32) + rd_ref[r]
        out_ref[...] = z


@jax.jit
def _finale_tc(r0, r1, r2, r3, batch3, ra, rc, rb, rd):
    grid = (N // BR,)
    bs_rep = pl.BlockSpec((2, BR, HALF), lambda i: (0, i, 0))
    return pl.pallas_call(
        _finale_kernel,
        grid=grid,
        in_specs=[
            bs_rep, bs_rep, bs_rep, bs_rep,
            pl.BlockSpec((1, 1, BR), lambda i: (i, 0, 0)),
            pl.BlockSpec((4, DH, DH), lambda i: (0, 0, 0)),
            pl.BlockSpec((4, 1, DH), lambda i: (0, 0, 0)),
            pl.BlockSpec((4, DH, DOUT), lambda i: (0, 0, 0)),
            pl.BlockSpec((4, 1, DOUT), lambda i: (0, 0, 0)),
        ],
        out_specs=pl.BlockSpec((G, DOUT), lambda i: (0, 0)),
        out_shape=jax.ShapeDtypeStruct((G, DOUT), jnp.float32),
        scratch_shapes=[
            pltpu.VMEM((4, G, DH), jnp.float32),
            pltpu.VMEM((G, HALF), jnp.float32),
        ],
    )(r0, r1, r2, r3, batch3, ra, rc, rb, rd)


import numpy as _np

def _wpack(wrow):
    r = (lax.bitcast_convert_type(wrow, jnp.uint32)
         + jnp.uint32(0x8000)) >> jnp.uint32(16)
    rr = r.reshape(4, 2, 16)
    return lax.bitcast_convert_type(
        (rr[:, 1, :] << jnp.uint32(16)) | rr[:, 0, :], jnp.int32).reshape(64)


def kernel(x, edge_index, edge_attr, batch, params):
    src = edge_index[0].astype(jnp.int32)
    dst = edge_index[1].astype(jnp.int32)
    def _attr_pack(a):
        r = (lax.bitcast_convert_type(a, jnp.uint32)
             + jnp.uint32(0x8000)) >> jnp.uint32(16)
        return lax.bitcast_convert_type((r << jnp.uint32(16)) | r, jnp.int32)

    a0 = _attr_pack(edge_attr[:, 0])
    a1 = _attr_pack(edge_attr[:, 1])
    zin = jnp.zeros((ROWS_PER_TILE, HALF), jnp.float32)
    batch3 = batch.astype(jnp.int32).reshape(N // BR, 1, BR)

    wb0 = params['convs'][0]['lin_edge'][1].reshape(1, DIN)
    h2, pk = _prep_tc(x, wb0)
    reps = [h2]
    for i in range(L):
        cp = params['convs'][i]
        Wl, _ = cp['lin_edge']
        wbf = jnp.stack([
            jnp.stack([_wpack(Wl[0, :HALF]), _wpack(Wl[1, :HALF])]),
            jnp.stack([_wpack(Wl[0, HALF:]), _wpack(Wl[1, HALF:])]),
        ])
        pkf = lax.bitcast_convert_type(pk, jnp.float32)
        agg2 = _edge_agg(pkf, src, dst, a0, a1, wbf, zin)
        (W1, b1), (W2, b2), (W3, b3) = cp['mlp']
        has_next = i < L - 1
        wbn = (params['convs'][i + 1]['lin_edge'][1] if has_next
               else jnp.zeros((DH,), jnp.float32)).reshape(1, DH)
        h2, pk = _layer_tc(h2, agg2, i == 0, has_next, cp['eps'].reshape(1, 1),
                           W1, b1.reshape(1, DH), W2, b2.reshape(1, DH),
                           W3, b3.reshape(1, DH), wbn)
        reps.append(h2)

    ra = jnp.stack([params['readouts'][i][0][0] for i in range(4)])
    rc = jnp.stack([params['readouts'][i][0][1].reshape(1, DH) for i in range(4)])
    rb = jnp.stack([params['readouts'][i][1][0] for i in range(4)])
    rd = jnp.stack([params['readouts'][i][1][1].reshape(1, DOUT) for i in range(4)])
    return _finale_tc(reps[0], reps[1], reps[2], reps[3], batch3, ra, rc, rb, rd)
